# Initial kernel scaffold; baseline (speedup 1.0000x reference)
import jax
import jax.numpy as jnp
from jax.experimental import pallas as pl


def kernel(node_features, edge_index, edge_probs, edge_params):
    raise NotImplementedError("write your pallas kernel here")


# speedup vs baseline: 104.4651x; 104.4651x over previous
import functools

import jax
import jax.numpy as jnp
from jax import lax
from jax.experimental import pallas as pl
from jax.experimental.pallas import tpu as pltpu
from jax.experimental.pallas import tpu_sc as plsc

N_NODES = 10000
N_EDGES = 640000
NC = 2
NS = 16
L = 16
NW = NC * NS
NPAD = 10240
RN = NPAD // 128
EPW = N_EDGES // NW
CH = 4000

_mesh = plsc.VectorSubcoreMesh(
    core_axis_name="c", subcore_axis_name="s", num_cores=NC, num_subcores=NS
)


@functools.partial(
    pl.kernel,
    out_type=(
        jax.ShapeDtypeStruct((NW, NPAD), jnp.float32),
        jax.ShapeDtypeStruct((NW, NPAD), jnp.float32),
    ),
    mesh=_mesh,
    compiler_params=pltpu.CompilerParams(needs_layout_passes=False),
    scratch_types=[
        pltpu.VMEM((NPAD,), jnp.float32),
        pltpu.VMEM((NPAD,), jnp.float32),
        pltpu.VMEM((NPAD,), jnp.float32),
        pltpu.VMEM((NPAD,), jnp.float32),
        pltpu.VMEM((NPAD,), jnp.float32),
        pltpu.VMEM((CH,), jnp.int32),
        pltpu.VMEM((CH,), jnp.int32),
        pltpu.VMEM((CH,), jnp.float32),
        pltpu.VMEM((CH,), jnp.float32),
        pltpu.VMEM((CH,), jnp.float32),
    ],
)
def _sc_flows(x_hbm, y_hbm, src_hbm, dst_hbm, prob_hbm, g_hbm, b_hbm,
              outp_hbm, outq_hbm,
              table, accp, accq, xbuf, ybuf, srcv, dstv, probv, gv, bv):
    cid = lax.axis_index("c")
    sid = lax.axis_index("s")
    wid = sid * NC + cid

    pltpu.sync_copy(x_hbm, xbuf)
    pltpu.sync_copy(y_hbm, ybuf)
    zeros = jnp.zeros((L,), jnp.float32)

    def fill(i, carry):
        xs = xbuf[pl.ds(i * L, L)]
        ys = ybuf[pl.ds(i * L, L)]
        table[pl.ds(i * L, L)] = xs * xs + ys * ys
        accp[pl.ds(i * L, L)] = zeros
        accq[pl.ds(i * L, L)] = zeros
        return carry

    lax.fori_loop(0, NPAD // L, fill, 0)

    ebase = wid * EPW

    def chunk_body(c, carry):
        off = ebase + c * CH
        pltpu.sync_copy(src_hbm.at[pl.ds(off, CH)], srcv)
        pltpu.sync_copy(dst_hbm.at[pl.ds(off, CH)], dstv)
        pltpu.sync_copy(prob_hbm.at[pl.ds(off, CH)], probv)
        pltpu.sync_copy(g_hbm.at[pl.ds(off, CH)], gv)
        pltpu.sync_copy(b_hbm.at[pl.ds(off, CH)], bv)

        def vec_body(i, c2):
            o = i * L
            s = srcv[pl.ds(o, L)]
            d = dstv[pl.ds(o, L)]
            v2 = plsc.load_gather(table, [s])
            vp = v2 * probv[pl.ds(o, L)]
            pe = vp / (gv[pl.ds(o, L)] + 1e-6)
            qe = vp / (bv[pl.ds(o, L)] + 1e-6)
            plsc.addupdate_scatter(accp, [s], pe)
            plsc.addupdate_scatter(accq, [s], qe)
            m = s != d
            plsc.addupdate_scatter(accp, [d], pe, mask=m)
            plsc.addupdate_scatter(accq, [d], qe, mask=m)
            return c2

        lax.fori_loop(0, CH // L, vec_body, 0)
        return carry

    lax.fori_loop(0, EPW // CH, chunk_body, 0)

    pltpu.sync_copy(accp, outp_hbm.at[wid])
    pltpu.sync_copy(accq, outq_hbm.at[wid])


def _tc_loss_body(p_ref, q_ref, x_ref, y_ref, pl_ref, ql_ref, o_ref):
    pf = jnp.sum(p_ref[...], axis=0)
    qf = jnp.sum(q_ref[...], axis=0)
    x = x_ref[...]
    y = y_ref[...]
    v = jnp.sqrt(x * x + y * y)
    pim = (pl_ref[...] + pf) ** 2
    qim = (ql_ref[...] + qf) ** 2
    lo = jnp.maximum(0.95 - v, 0.0)
    hi = jnp.maximum(v - 1.05, 0.0)
    row = lax.broadcasted_iota(jnp.int32, (RN, 128), 0)
    col = lax.broadcasted_iota(jnp.int32, (RN, 128), 1)
    valid = (row * 128 + col) < N_NODES
    tot = jnp.sum(jnp.where(valid, pim + qim + lo * lo + hi * hi, 0.0))
    o_ref[0, 0] = tot / N_NODES


_tc_loss = pl.pallas_call(
    _tc_loss_body,
    out_shape=jax.ShapeDtypeStruct((1, 1), jnp.float32),
    out_specs=pl.BlockSpec(memory_space=pltpu.SMEM),
)


def kernel(node_features, edge_index, edge_probs, edge_params):
    src = edge_index[0].astype(jnp.int32)
    dst = edge_index[1].astype(jnp.int32)
    pad = NPAD - N_NODES
    xp = jnp.pad(node_features[:, 0], (0, pad))
    yp = jnp.pad(node_features[:, 1], (0, pad))
    plp = jnp.pad(node_features[:, 2], (0, pad))
    qlp = jnp.pad(node_features[:, 3], (0, pad))
    g = edge_params[:, 0]
    b = edge_params[:, 1]

    p_parts, q_parts = _sc_flows(xp, yp, src, dst, edge_probs, g, b)

    out = _tc_loss(
        p_parts.reshape(NW, RN, 128),
        q_parts.reshape(NW, RN, 128),
        xp.reshape(RN, 128),
        yp.reshape(RN, 128),
        plp.reshape(RN, 128),
        qlp.reshape(RN, 128),
    )
    return out[0, 0]
